# baseline (speedup 1.0000x reference)
import functools

import jax
import jax.numpy as jnp
from jax import lax
from jax.experimental import pallas as pl
from jax.experimental.pallas import tpu as pltpu
from jax.experimental.pallas import tpu_sc as plsc

_B, _N, _M = 8, 2048, 2048
_NC, _NS = 2, 16
_NW = _NC * _NS
_CPB = _NW // _B
_CHUNK = _N // _CPB
_GB = 4
_NGB = _CHUNK // (_GB * 16)


def _min_pass(b, ck, src_chunk, src_full, chunk_v, full_v):
    for d in range(3):
        pltpu.sync_copy(src_chunk[d].at[pl.ds(b * _N + ck * _CHUNK, _CHUNK)],
                        chunk_v[d])
        pltpu.sync_copy(src_full[d].at[pl.ds(b * _N, _N)], full_v[d])

    def gbody(g, total):
        base = g * _GB * 16
        px = [chunk_v[0][pl.ds(base + k * 16, 16)] for k in range(_GB)]
        py = [chunk_v[1][pl.ds(base + k * 16, 16)] for k in range(_GB)]
        pz = [chunk_v[2][pl.ds(base + k * 16, 16)] for k in range(_GB)]
        init = tuple(jnp.full((16,), jnp.inf, jnp.float32) for _ in range(_GB))

        def jbody(jb, mins):
            txv = full_v[0][pl.ds(jb * 16, 16)]
            tyv = full_v[1][pl.ds(jb * 16, 16)]
            tzv = full_v[2][pl.ds(jb * 16, 16)]
            mins = list(mins)
            for jj in range(16):
                tx = txv[jj]
                ty = tyv[jj]
                tz = tzv[jj]
                for k in range(_GB):
                    dx = px[k] - tx
                    dy = py[k] - ty
                    dz = pz[k] - tz
                    d2 = dx * dx + dy * dy + dz * dz
                    mins[k] = jnp.minimum(mins[k], d2)
            return tuple(mins)

        mins = lax.fori_loop(0, _M // 16, jbody, init)
        for k in range(_GB):
            total = total + mins[k]
        return total

    return lax.fori_loop(0, _NGB, gbody, jnp.zeros((16,), jnp.float32))


@functools.partial(
    pl.kernel,
    out_type=jax.ShapeDtypeStruct((_NW * 16,), jnp.float32),
    mesh=plsc.VectorSubcoreMesh(core_axis_name="c", subcore_axis_name="s"),
    scratch_types=[
        pltpu.VMEM((_CHUNK,), jnp.float32),
        pltpu.VMEM((_CHUNK,), jnp.float32),
        pltpu.VMEM((_CHUNK,), jnp.float32),
        pltpu.VMEM((_N,), jnp.float32),
        pltpu.VMEM((_N,), jnp.float32),
        pltpu.VMEM((_N,), jnp.float32),
        pltpu.VMEM((16,), jnp.float32),
    ],
)
def _sc_chamfer(px_h, py_h, pz_h, tx_h, ty_h, tz_h, out_hbm,
                cx, cy, cz, fx, fy, fz, sum_v):
    chunk_v = (cx, cy, cz)
    full_v = (fx, fy, fz)
    pred_h = (px_h, py_h, pz_h)
    targ_h = (tx_h, ty_h, tz_h)
    wid = lax.axis_index("s") * _NC + lax.axis_index("c")
    b = wid // _CPB
    ck = wid % _CPB
    tot_a = _min_pass(b, ck, pred_h, targ_h, chunk_v, full_v)
    tot_b = _min_pass(b, ck, targ_h, pred_h, chunk_v, full_v)
    sum_v[...] = (tot_a + tot_b) * (1.0 / (_B * _N))
    pltpu.sync_copy(sum_v, out_hbm.at[pl.ds(wid * 16, 16)])


def kernel(pred, target):
    pred = pred.astype(jnp.float32)
    target = target.astype(jnp.float32)
    coords_p = [pred[:, :, d].reshape(-1) for d in range(3)]
    coords_t = [target[:, :, d].reshape(-1) for d in range(3)]
    parts = _sc_chamfer(*coords_p, *coords_t)
    return jnp.sum(parts)


# speedup vs baseline: 11.0335x; 11.0335x over previous
import functools

import jax
import jax.numpy as jnp
from jax import lax
from jax.experimental import pallas as pl
from jax.experimental.pallas import tpu as pltpu
from jax.experimental.pallas import tpu_sc as plsc

_B, _N = 8, 2048
_PAD = 16
_NP = _N + _PAD
_NC, _NS = 2, 16
_NW = _NC * _NS
_BPC = _B // _NC
_CPB = _NS // _BPC
_CHUNK = _N // _CPB
_GB = 4
_NGB = _CHUNK // (_GB * 16)
_INF = 3.4e38


@functools.partial(
    pl.kernel,
    out_type=jax.ShapeDtypeStruct((_NW * 16,), jnp.float32),
    mesh=plsc.VectorSubcoreMesh(core_axis_name="c", subcore_axis_name="s"),
    scratch_types=[
        pltpu.VMEM((_CHUNK,), jnp.float32),
        pltpu.VMEM((_CHUNK,), jnp.float32),
        pltpu.VMEM((_CHUNK,), jnp.float32),
        pltpu.VMEM((_NP,), jnp.float32),
        pltpu.VMEM((_NP,), jnp.float32),
        pltpu.VMEM((_NP,), jnp.float32),
        pltpu.VMEM((_NP,), jnp.float32),
        pltpu.VMEM((_CPB * _CHUNK,), jnp.float32),
        pltpu.VMEM((16,), jnp.float32),
        pltpu.VMEM_SHARED((_NS * _N,), jnp.float32),
    ],
)
def _sc_chamfer(px_h, py_h, pz_h, tx_h, ty_h, tz_h, out_hbm,
                cx, cy, cz, fx, fy, fz, d2min, comb, sum_v, shared):
    c = lax.axis_index("c")
    s = lax.axis_index("s")
    b = c * _BPC + s // _CPB
    ck = s % _CPB
    wid = c * _NS + s

    cbase = b * _NP + ck * _CHUNK
    for src, dst in ((px_h, cx), (py_h, cy), (pz_h, cz)):
        pltpu.sync_copy(src.at[pl.ds(cbase, _CHUNK)], dst)
    for src, dst in ((tx_h, fx), (ty_h, fy), (tz_h, fz)):
        pltpu.sync_copy(src.at[pl.ds(b * _NP, _NP)], dst)

    inf_v = jnp.full((16,), _INF, jnp.float32)

    def initbody(i, carry):
        d2min[pl.ds(i * 16, 16)] = inf_v
        return carry

    lax.fori_loop(0, _NP // 16, initbody, 0)

    def gbody(g, total1):
        base = g * _GB * 16
        px = [cx[pl.ds(base + k * 16, 16)] for k in range(_GB)]
        py = [cy[pl.ds(base + k * 16, 16)] for k in range(_GB)]
        pz = [cz[pl.ds(base + k * 16, 16)] for k in range(_GB)]
        init = tuple(jnp.full((16,), _INF, jnp.float32) for k in range(_GB))

        def obody(o, mins):
            tx = fx[pl.ds(o, 16)]
            ty = fy[pl.ds(o, 16)]
            tz = fz[pl.ds(o, 16)]
            new = []
            d2s = []
            for k in range(_GB):
                dx = px[k] - tx
                dy = py[k] - ty
                dz = pz[k] - tz
                d2 = dx * dx + dy * dy + dz * dz
                d2s.append(d2)
                new.append(jnp.minimum(mins[k], d2))
            colmin = jnp.minimum(jnp.minimum(d2min[pl.ds(o, 16)],
                                             jnp.minimum(d2s[0], d2s[1])),
                                 jnp.minimum(d2s[2], d2s[3]))
            d2min[pl.ds(o, 16)] = colmin
            return tuple(new)

        mins = lax.fori_loop(0, _N, obody, init)
        for k in range(_GB):
            total1 = total1 + mins[k]
        return total1

    total1 = lax.fori_loop(0, _NGB, gbody, jnp.zeros((16,), jnp.float32))

    d2min[pl.ds(0, 16)] = jnp.minimum(d2min[pl.ds(0, 16)],
                                      d2min[pl.ds(_N, 16)])

    pltpu.sync_copy(d2min.at[pl.ds(0, _N)], shared.at[pl.ds(s * _N, _N)])
    plsc.subcore_barrier()
    s0 = (s // _CPB) * _CPB
    q = s % _CPB
    for r in range(_CPB):
        pltpu.sync_copy(
            shared.at[pl.ds((s0 + r) * _N + q * _CHUNK, _CHUNK)],
            comb.at[pl.ds(r * _CHUNK, _CHUNK)])

    def combbody(i, total2):
        m01 = jnp.minimum(comb[pl.ds(i * 16, 16)],
                          comb[pl.ds(_CHUNK + i * 16, 16)])
        m23 = jnp.minimum(comb[pl.ds(2 * _CHUNK + i * 16, 16)],
                          comb[pl.ds(3 * _CHUNK + i * 16, 16)])
        return total2 + jnp.minimum(m01, m23)

    total2 = lax.fori_loop(0, _CHUNK // 16, combbody,
                           jnp.zeros((16,), jnp.float32))

    sum_v[...] = (total1 + total2) * (1.0 / (_B * _N))
    pltpu.sync_copy(sum_v, out_hbm.at[pl.ds(wid * 16, 16)])


def kernel(pred, target):
    pred = pred.astype(jnp.float32)
    target = target.astype(jnp.float32)
    predp = jnp.concatenate([pred, pred[:, :_PAD]], axis=1)
    targp = jnp.concatenate([target, target[:, :_PAD]], axis=1)
    coords_p = [predp[:, :, d].reshape(-1) for d in range(3)]
    coords_t = [targp[:, :, d].reshape(-1) for d in range(3)]
    parts = _sc_chamfer(*coords_p, *coords_t)
    return jnp.sum(parts)


# speedup vs baseline: 13.9791x; 1.2670x over previous
import jax
import jax.numpy as jnp
from jax.experimental import pallas as pl
from jax.experimental.pallas import tpu as pltpu

_B, _N, _M = 8, 2048, 2048
_K = 8


def _chamfer_body(ps_ref, tT_ref, out_ref):
    b = pl.program_id(0)
    ps = ps_ref[0]
    tT = tT_ref[0]
    pn2 = jnp.sum(ps * ps, axis=1, keepdims=True) * 0.25
    tm2 = jnp.sum(tT * tT, axis=0, keepdims=True)
    d2 = jax.lax.dot(ps, tT, preferred_element_type=jnp.float32,
                     precision=jax.lax.Precision.HIGHEST)
    d2 = d2 + pn2 + tm2
    s1 = jnp.sum(jnp.min(d2, axis=1))
    s2 = jnp.sum(jnp.min(d2, axis=0))

    @pl.when(b == 0)
    def _():
        out_ref[0, 0] = 0.0

    out_ref[0, 0] += (s1 + s2) * (1.0 / (_B * _N))


def kernel(pred, target):
    pred = pred.astype(jnp.float32)
    target = target.astype(jnp.float32)
    ps = jnp.pad(pred * -2.0, ((0, 0), (0, 0), (0, _K - 3)))
    tT = jnp.pad(target, ((0, 0), (0, 0), (0, _K - 3))).swapaxes(1, 2)
    out = pl.pallas_call(
        _chamfer_body,
        grid=(_B,),
        in_specs=[
            pl.BlockSpec((1, _N, _K), lambda b: (b, 0, 0)),
            pl.BlockSpec((1, _K, _M), lambda b: (b, 0, 0)),
        ],
        out_specs=pl.BlockSpec(memory_space=pltpu.SMEM),
        out_shape=jax.ShapeDtypeStruct((1, 1), jnp.float32),
        compiler_params=pltpu.CompilerParams(
            dimension_semantics=("arbitrary",),
        ),
    )(ps, tT)
    return out[0, 0]


# speedup vs baseline: 24.3740x; 1.7436x over previous
import functools

import jax
import jax.numpy as jnp
from jax import lax
from jax.experimental import pallas as pl
from jax.experimental.pallas import tpu as pltpu
from jax.experimental.pallas import tpu_sc as plsc

_B, _N, _M = 8, 2048, 2048
_SCALE = 1.0 / (_B * _N)


_BSC = 2
_PAD = 16
_NP = _N + _PAD
_NC, _NS = 2, 16
_NW = _NC * _NS
_CPB = _NS
_CHUNK = _N // _CPB
_GB = 4
_NGB = _CHUNK // (_GB * 16)
_QT = _N // _CPB
_INF = 3.4e38


@functools.partial(
    pl.kernel,
    out_type=jax.ShapeDtypeStruct((_NW * 16,), jnp.float32),
    mesh=plsc.VectorSubcoreMesh(core_axis_name="c", subcore_axis_name="s"),
    scratch_types=[
        pltpu.VMEM((_CHUNK,), jnp.float32),
        pltpu.VMEM((_CHUNK,), jnp.float32),
        pltpu.VMEM((_CHUNK,), jnp.float32),
        pltpu.VMEM((_NP,), jnp.float32),
        pltpu.VMEM((_NP,), jnp.float32),
        pltpu.VMEM((_NP,), jnp.float32),
        pltpu.VMEM((_NP,), jnp.float32),
        pltpu.VMEM((_CPB * _QT,), jnp.float32),
        pltpu.VMEM((16,), jnp.float32),
        pltpu.VMEM_SHARED((_NS * _N,), jnp.float32),
    ],
)
def _sc_chamfer(px_h, py_h, pz_h, tx_h, ty_h, tz_h, out_hbm,
                cx, cy, cz, fx, fy, fz, d2min, comb, sum_v, shared):
    c = lax.axis_index("c")
    s = lax.axis_index("s")
    b = c
    ck = s
    wid = c * _NS + s

    cbase = b * _NP + ck * _CHUNK
    for src, dst in ((px_h, cx), (py_h, cy), (pz_h, cz)):
        pltpu.sync_copy(src.at[pl.ds(cbase, _CHUNK)], dst)
    for src, dst in ((tx_h, fx), (ty_h, fy), (tz_h, fz)):
        pltpu.sync_copy(src.at[pl.ds(b * _NP, _NP)], dst)

    inf_v = jnp.full((16,), _INF, jnp.float32)

    def initbody(i, carry):
        d2min[pl.ds(i * 16, 16)] = inf_v
        return carry

    lax.fori_loop(0, _NP // 16, initbody, 0)

    def gbody(g, total1):
        base = g * _GB * 16
        px = [cx[pl.ds(base + k * 16, 16)] for k in range(_GB)]
        py = [cy[pl.ds(base + k * 16, 16)] for k in range(_GB)]
        pz = [cz[pl.ds(base + k * 16, 16)] for k in range(_GB)]
        init = tuple(jnp.full((16,), _INF, jnp.float32) for k in range(_GB))

        def obody(o, mins):
            tx = fx[pl.ds(o, 16)]
            ty = fy[pl.ds(o, 16)]
            tz = fz[pl.ds(o, 16)]
            new = []
            d2s = []
            for k in range(_GB):
                dx = px[k] - tx
                dy = py[k] - ty
                dz = pz[k] - tz
                d2 = dx * dx + dy * dy + dz * dz
                d2s.append(d2)
                new.append(jnp.minimum(mins[k], d2))
            colmin = jnp.minimum(jnp.minimum(d2min[pl.ds(o, 16)],
                                             jnp.minimum(d2s[0], d2s[1])),
                                 jnp.minimum(d2s[2], d2s[3]))
            d2min[pl.ds(o, 16)] = colmin
            return tuple(new)

        mins = lax.fori_loop(0, _N, obody, init)
        for k in range(_GB):
            total1 = total1 + mins[k]
        return total1

    total1 = lax.fori_loop(0, _NGB, gbody, jnp.zeros((16,), jnp.float32))

    d2min[pl.ds(0, 16)] = jnp.minimum(d2min[pl.ds(0, 16)],
                                      d2min[pl.ds(_N, 16)])

    pltpu.sync_copy(d2min.at[pl.ds(0, _N)], shared.at[pl.ds(s * _N, _N)])
    plsc.subcore_barrier()
    for r in range(_CPB):
        pltpu.sync_copy(shared.at[pl.ds(r * _N + s * _QT, _QT)],
                        comb.at[pl.ds(r * _QT, _QT)])

    def combbody(i, total2):
        m = comb[pl.ds(i * 16, 16)]
        for r in range(1, _CPB):
            m = jnp.minimum(m, comb[pl.ds(r * _QT + i * 16, 16)])
        return total2 + m

    total2 = lax.fori_loop(0, _QT // 16, combbody,
                           jnp.zeros((16,), jnp.float32))

    sum_v[...] = (total1 + total2) * _SCALE
    pltpu.sync_copy(sum_v, out_hbm.at[pl.ds(wid * 16, 16)])


_BTC = _B - _BSC


def _tc_body(p_ref, tT_ref, out_ref):
    b = pl.program_id(0)
    p = p_ref[0]
    tT = tT_ref[0]
    d2 = (p[:, 0:1] - tT[0:1, :]) ** 2
    d2 += (p[:, 1:2] - tT[1:2, :]) ** 2
    d2 += (p[:, 2:3] - tT[2:3, :]) ** 2
    s1 = jnp.sum(jnp.min(d2, axis=1))
    s2 = jnp.sum(jnp.min(d2, axis=0))

    @pl.when(b == 0)
    def _():
        out_ref[0, 0] = 0.0

    out_ref[0, 0] += (s1 + s2) * _SCALE


def _tc_chamfer(pred, target):
    tT = target.swapaxes(1, 2)
    out = pl.pallas_call(
        _tc_body,
        grid=(_BTC,),
        in_specs=[
            pl.BlockSpec((1, _N, 3), lambda b: (b, 0, 0)),
            pl.BlockSpec((1, 3, _M), lambda b: (b, 0, 0)),
        ],
        out_specs=pl.BlockSpec(memory_space=pltpu.SMEM),
        out_shape=jax.ShapeDtypeStruct((1, 1), jnp.float32),
        compiler_params=pltpu.CompilerParams(
            dimension_semantics=("arbitrary",),
        ),
    )(pred, tT)
    return out[0, 0]


def kernel(pred, target):
    pred = pred.astype(jnp.float32)
    target = target.astype(jnp.float32)
    predp = jnp.concatenate([pred[:_BSC], pred[:_BSC, :_PAD]], axis=1)
    targp = jnp.concatenate([target[:_BSC], target[:_BSC, :_PAD]], axis=1)
    coords_p = [predp[:, :, d].reshape(-1) for d in range(3)]
    coords_t = [targp[:, :, d].reshape(-1) for d in range(3)]
    sc_parts = _sc_chamfer(*coords_p, *coords_t)
    tc_part = _tc_chamfer(pred[_BSC:], target[_BSC:])
    return jnp.sum(sc_parts) + tc_part


# speedup vs baseline: 25.7425x; 1.0561x over previous
import functools

import jax
import jax.numpy as jnp
from jax import lax
from jax.experimental import pallas as pl
from jax.experimental.pallas import tpu as pltpu
from jax.experimental.pallas import tpu_sc as plsc

_B, _N, _M = 8, 2048, 2048
_SCALE = 1.0 / (_B * _N)


_BSC = 2
_PAD = 16
_NP = _N + _PAD
_NC, _NS = 2, 16
_NW = _NC * _NS
_CPB = _NS
_CHUNK = _N // _CPB
_GB = 4
_NGB = _CHUNK // (_GB * 16)
_QT = _N // _CPB
_INF = 3.4e38


@functools.partial(
    pl.kernel,
    out_type=jax.ShapeDtypeStruct((_NW * 16,), jnp.float32),
    mesh=plsc.VectorSubcoreMesh(core_axis_name="c", subcore_axis_name="s"),
    scratch_types=[
        pltpu.VMEM((_CHUNK,), jnp.float32),
        pltpu.VMEM((_CHUNK,), jnp.float32),
        pltpu.VMEM((_CHUNK,), jnp.float32),
        pltpu.VMEM((_NP,), jnp.float32),
        pltpu.VMEM((_NP,), jnp.float32),
        pltpu.VMEM((_NP,), jnp.float32),
        pltpu.VMEM((_NP,), jnp.float32),
        pltpu.VMEM((_CPB * _QT,), jnp.float32),
        pltpu.VMEM((16,), jnp.float32),
        pltpu.VMEM_SHARED((_NS * _N,), jnp.float32),
    ],
)
def _sc_chamfer(flat_h, out_hbm,
                cx, cy, cz, fx, fy, fz, d2min, comb, sum_v, shared):
    c = lax.axis_index("c")
    s = lax.axis_index("s")
    b = c
    ck = s
    wid = c * _NS + s

    for d, dst in enumerate((cx, cy, cz)):
        off = (d * _BSC + b) * _NP + ck * _CHUNK
        pltpu.sync_copy(flat_h.at[pl.ds(off, _CHUNK)], dst)
    for d, dst in enumerate((fx, fy, fz)):
        off = ((3 + d) * _BSC + b) * _NP
        pltpu.sync_copy(flat_h.at[pl.ds(off, _NP)], dst)

    inf_v = jnp.full((16,), _INF, jnp.float32)

    def initbody(i, carry):
        d2min[pl.ds(i * 16, 16)] = inf_v
        return carry

    lax.fori_loop(0, _NP // 16, initbody, 0)

    def gbody(g, total1):
        base = g * _GB * 16
        px = [cx[pl.ds(base + k * 16, 16)] for k in range(_GB)]
        py = [cy[pl.ds(base + k * 16, 16)] for k in range(_GB)]
        pz = [cz[pl.ds(base + k * 16, 16)] for k in range(_GB)]
        init = tuple(jnp.full((16,), _INF, jnp.float32) for k in range(_GB))

        def obody(o, mins):
            tx = fx[pl.ds(o, 16)]
            ty = fy[pl.ds(o, 16)]
            tz = fz[pl.ds(o, 16)]
            new = []
            d2s = []
            for k in range(_GB):
                dx = px[k] - tx
                dy = py[k] - ty
                dz = pz[k] - tz
                d2 = dx * dx + dy * dy + dz * dz
                d2s.append(d2)
                new.append(jnp.minimum(mins[k], d2))
            colmin = jnp.minimum(jnp.minimum(d2min[pl.ds(o, 16)],
                                             jnp.minimum(d2s[0], d2s[1])),
                                 jnp.minimum(d2s[2], d2s[3]))
            d2min[pl.ds(o, 16)] = colmin
            return tuple(new)

        mins = lax.fori_loop(0, _N, obody, init)
        for k in range(_GB):
            total1 = total1 + mins[k]
        return total1

    total1 = lax.fori_loop(0, _NGB, gbody, jnp.zeros((16,), jnp.float32))

    d2min[pl.ds(0, 16)] = jnp.minimum(d2min[pl.ds(0, 16)],
                                      d2min[pl.ds(_N, 16)])

    pltpu.sync_copy(d2min.at[pl.ds(0, _N)], shared.at[pl.ds(s * _N, _N)])
    plsc.subcore_barrier()
    for r in range(_CPB):
        pltpu.sync_copy(shared.at[pl.ds(r * _N + s * _QT, _QT)],
                        comb.at[pl.ds(r * _QT, _QT)])

    def combbody(i, total2):
        m = comb[pl.ds(i * 16, 16)]
        for r in range(1, _CPB):
            m = jnp.minimum(m, comb[pl.ds(r * _QT + i * 16, 16)])
        return total2 + m

    total2 = lax.fori_loop(0, _QT // 16, combbody,
                           jnp.zeros((16,), jnp.float32))

    sum_v[...] = (total1 + total2) * _SCALE
    pltpu.sync_copy(sum_v, out_hbm.at[pl.ds(wid * 16, 16)])


_BTC = _B - _BSC


def _tc_body(p_ref, tT_ref, out_ref):
    b = pl.program_id(0)
    p = p_ref[0]
    tT = tT_ref[0]
    d2 = (p[:, 0:1] - tT[0:1, :]) ** 2
    d2 += (p[:, 1:2] - tT[1:2, :]) ** 2
    d2 += (p[:, 2:3] - tT[2:3, :]) ** 2
    s1 = jnp.sum(jnp.min(d2, axis=1))
    s2 = jnp.sum(jnp.min(d2, axis=0))

    @pl.when(b == 0)
    def _():
        out_ref[0, 0] = 0.0

    out_ref[0, 0] += (s1 + s2) * _SCALE


def _tc_chamfer(pred, tT):
    out = pl.pallas_call(
        _tc_body,
        grid=(_BTC,),
        in_specs=[
            pl.BlockSpec((1, _N, 3), lambda b: (b + _BSC, 0, 0)),
            pl.BlockSpec((1, 3, _M), lambda b: (b + _BSC, 0, 0)),
        ],
        out_specs=pl.BlockSpec(memory_space=pltpu.SMEM),
        out_shape=jax.ShapeDtypeStruct((1, 1), jnp.float32),
        compiler_params=pltpu.CompilerParams(
            dimension_semantics=("arbitrary",),
        ),
    )(pred, tT)
    return out[0, 0]


def kernel(pred, target):
    pred = pred.astype(jnp.float32)
    target = target.astype(jnp.float32)
    pt = jnp.stack([pred[:_BSC], target[:_BSC]])
    ptp = jnp.concatenate([pt, pt[:, :, :_PAD]], axis=2)
    flat = ptp.transpose(0, 3, 1, 2).reshape(-1)
    sc_parts = _sc_chamfer(flat)
    tc_part = _tc_chamfer(pred, target.swapaxes(1, 2))
    return jnp.sum(sc_parts) + tc_part
